# Initial kernel scaffold; baseline (speedup 1.0000x reference)
import jax
import jax.numpy as jnp
from jax.experimental import pallas as pl


def kernel(node_feats, edge_index, edge_weight, W1, b1, W2, b2):
    raise NotImplementedError("write your pallas kernel here")


# speedup vs baseline: 1.1861x; 1.1861x over previous
import functools

import jax
import jax.numpy as jnp
from jax import lax
from jax.experimental import pallas as pl
from jax.experimental.pallas import tpu as pltpu
from jax.experimental.pallas import tpu_sc as plsc

N_NODES = 10000
N_EDGES = 160000
F = 256
IN_F = 256
HID_F = 512
OUT_F = 256

NC = 2
NS = 16
NW = NC * NS
L = 16

NPAD = 10240
RPT = NPAD // NW
CH = 2000
NCHUNK = N_EDGES // CH
assert CH % L == 0 and N_EDGES % CH == 0
BATCH = 64
EPW = N_EDGES // NW

@functools.cache
def _mesh():
  return plsc.VectorSubcoreMesh(
      core_axis_name="c", subcore_axis_name="s", num_cores=NC, num_subcores=NS
  )


_SC_PARAMS = pltpu.CompilerParams(needs_layout_passes=False)


def _zero_vecs(ref, n16, dtype):
  z = jnp.zeros((L,), dtype)

  def body(i, _):
    ref[pl.ds(i * L, L)] = z
    return 0

  lax.fori_loop(0, n16, body, 0)


def _deg_body(src, dst, od, idg, sbuf, dbuf, hs, hd, tmp, acc_s, acc_d,
              sh_s, sh_d):
  c = lax.axis_index("c")
  s = lax.axis_index("s")
  wid = c * NS + s
  e0 = wid * EPW

  pltpu.sync_copy(src.at[pl.ds(e0, EPW)], sbuf.at[pl.ds(0, EPW)])
  pltpu.sync_copy(dst.at[pl.ds(e0, EPW)], dbuf.at[pl.ds(0, EPW)])

  _zero_vecs(hs, NPAD // L, jnp.int32)
  _zero_vecs(hd, NPAD // L, jnp.int32)

  ones_i = jnp.ones((L,), jnp.int32)
  nfull = EPW // L
  rem = EPW - nfull * L

  def inc(v, _):
    ks = sbuf[pl.ds(v * L, L)]
    kd = dbuf[pl.ds(v * L, L)]
    plsc.addupdate_scatter(hs, [ks], ones_i)
    plsc.addupdate_scatter(hd, [kd], ones_i)
    return 0

  lax.fori_loop(0, nfull, inc, 0)
  if rem:
    mtail = jnp.arange(L, dtype=jnp.int32) < rem
    sbuf[pl.ds(EPW, L)] = jnp.zeros((L,), jnp.int32)
    dbuf[pl.ds(EPW, L)] = jnp.zeros((L,), jnp.int32)
    ks = sbuf[pl.ds(nfull * L, L)]
    kd = dbuf[pl.ds(nfull * L, L)]
    plsc.addupdate_scatter(hs, [ks], ones_i, mask=mtail)
    plsc.addupdate_scatter(hd, [kd], ones_i, mask=mtail)

  pltpu.sync_copy(hs, sh_s.at[s])
  pltpu.sync_copy(hd, sh_d.at[s])
  plsc.subcore_barrier()

  col0 = s * (NPAD // NS)
  nv = (NPAD // NS) // L
  _zero_vecs(acc_s, nv, jnp.int32)
  _zero_vecs(acc_d, nv, jnp.int32)

  def addv(accref):
    def body(i, _):
      accref[pl.ds(i * L, L)] = accref[pl.ds(i * L, L)] + tmp[pl.ds(i * L, L)]
      return 0
    lax.fori_loop(0, nv, body, 0)

  for k in range(NS):
    pltpu.sync_copy(sh_s.at[k, pl.ds(col0, NPAD // NS)], tmp)
    addv(acc_s)
    pltpu.sync_copy(sh_d.at[k, pl.ds(col0, NPAD // NS)], tmp)
    addv(acc_d)

  pltpu.sync_copy(acc_s, od.at[c, pl.ds(col0, NPAD // NS)])
  pltpu.sync_copy(acc_d, idg.at[c, pl.ds(col0, NPAD // NS)])


@functools.partial(jax.jit, static_argnums=())
def _deg_call(src, dst):
  return pl.kernel(
      _deg_body,
      out_type=(
          jax.ShapeDtypeStruct((NC, NPAD), jnp.int32),
          jax.ShapeDtypeStruct((NC, NPAD), jnp.int32),
      ),
      mesh=_mesh(),
      compiler_params=_SC_PARAMS,
      scratch_types=[
          pltpu.VMEM((EPW + L,), jnp.int32),
          pltpu.VMEM((EPW + L,), jnp.int32),
          pltpu.VMEM((NPAD,), jnp.int32),
          pltpu.VMEM((NPAD,), jnp.int32),
          pltpu.VMEM((NPAD // NS,), jnp.int32),
          pltpu.VMEM((NPAD // NS,), jnp.int32),
          pltpu.VMEM((NPAD // NS,), jnp.int32),
          pltpu.VMEM_SHARED((NS, NPAD), jnp.int32),
          pltpu.VMEM_SHARED((NS, NPAD), jnp.int32),
      ],
  )(src, dst)


def _norm_body(od_ref, id_ref, ns_ref, nd_ref):
  do = (od_ref[0] + od_ref[1]).astype(jnp.float32)
  di = (id_ref[0] + id_ref[1]).astype(jnp.float32)
  ns_ref[...] = lax.rsqrt(jnp.maximum(do, 1.0))
  nd_ref[...] = lax.rsqrt(jnp.maximum(di, 1.0))


def _norm_call(od, idg):
  out = pl.pallas_call(
      _norm_body,
      out_shape=(
          jax.ShapeDtypeStruct((NPAD // 128, 128), jnp.float32),
          jax.ShapeDtypeStruct((NPAD // 128, 128), jnp.float32),
      ),
  )(od.reshape(NC, NPAD // 128, 128), idg.reshape(NC, NPAD // 128, 128))
  return out[0].reshape(NPAD), out[1].reshape(NPAD)


def _scatter_body(table, src, dst, w, ns, nd, bias, out,
                  srcc, dstc, wc, csrc, cdst, cw, rows, acc,
                  nsv, ndv, biasv, sem):
  c = lax.axis_index("c")
  s = lax.axis_index("s")
  wid = c * NS + s
  lo = wid * RPT

  pltpu.sync_copy(ns, nsv)
  pltpu.sync_copy(nd.at[pl.ds(wid * RPT, RPT)], ndv.at[pl.ds(0, RPT)])
  pltpu.sync_copy(bias, biasv)

  _zero_vecs(acc, RPT * F // L, jnp.float32)

  padg = lax.rem(wid * 313 + 17, N_NODES)

  def chunk_body(k, _):
    e0 = k * CH
    pltpu.sync_copy(src.at[pl.ds(e0, CH)], srcc)
    pltpu.sync_copy(dst.at[pl.ds(e0, CH)], dstc)
    pltpu.sync_copy(w.at[pl.ds(e0, CH)], wc)

    pg = jnp.full((L,), 0, jnp.int32) + padg
    zi = jnp.zeros((L,), jnp.int32)
    zf = jnp.zeros((L,), jnp.float32)

    def pre(i, _):
      csrc[pl.ds(i * L, L)] = pg
      cdst[pl.ds(i * L, L)] = zi
      cw[pl.ds(i * L, L)] = zf
      return 0

    lax.fori_loop(0, (CH + BATCH + L) // L, pre, 0)

    def comp(v, cnt):
      kd = dstc[pl.ds(v * L, L)]
      ks = srcc[pl.ds(v * L, L)]
      kw = wc[pl.ds(v * L, L)]
      m = (kd >= lo) & (kd < lo + RPT)
      ld = kd - lo
      plsc.store_compressed(csrc.at[pl.ds(cnt, L)], ks, mask=m)
      plsc.store_compressed(cdst.at[pl.ds(cnt, L)], ld, mask=m)
      plsc.store_compressed(cw.at[pl.ds(cnt, L)], kw, mask=m)
      pc = plsc.all_reduce_population_count(m)
      return cnt + pc[0]

    cnt = lax.fori_loop(0, CH // L, comp, 0)

    def batch_body(b, _):
      copies = []
      for j in range(BATCH // L):
        cs = csrc[pl.ds(b * BATCH + j * L, L)]
        nsg = plsc.load_gather(nsv, [cs])
        cw[pl.ds(b * BATCH + j * L, L)] = cw[pl.ds(b * BATCH + j * L, L)] * nsg
        copies.append(
            pltpu.async_copy(table.at[cs], rows.at[pl.ds(j * L, L)], sem))
      for cp in copies:
        cp.wait()

      def srow(i, _):
        ws = cw[pl.ds(b * BATCH + i, L)][0]
        ld = cdst[pl.ds(b * BATCH + i, L)][0]
        base = ld * F
        for j in range(F // L):
          v = rows[i, pl.ds(j * L, L)] * ws
          plsc.addupdate(acc.at[pl.ds(base + j * L, L)], v)
        return 0

      lax.fori_loop(0, BATCH, srow, 0)
      return 0

    nb = (cnt + BATCH - 1) // BATCH
    lax.fori_loop(0, nb, batch_body, 0)
    return 0

  lax.fori_loop(0, NCHUNK, chunk_body, 0)

  for blk in range(RPT // BATCH):
    def drow(r, _):
      nds = ndv[pl.ds(blk * BATCH + r, L)][0]
      base = (blk * BATCH + r) * F
      for j in range(F // L):
        v = acc[pl.ds(base + j * L, L)]
        rows[r, pl.ds(j * L, L)] = v * nds + biasv[pl.ds(j * L, L)]
      return 0

    lax.fori_loop(0, BATCH, drow, 0)
    pltpu.sync_copy(rows, out.at[pl.ds(lo + blk * BATCH, BATCH)])


def _scatter_call(table, src, dst, w, ns, nd, bias):
  return pl.kernel(
      _scatter_body,
      out_type=jax.ShapeDtypeStruct((NPAD, F), jnp.float32),
      mesh=_mesh(),
      compiler_params=_SC_PARAMS,
      scratch_types=[
          pltpu.VMEM((CH,), jnp.int32),
          pltpu.VMEM((CH,), jnp.int32),
          pltpu.VMEM((CH,), jnp.float32),
          pltpu.VMEM((CH + BATCH + L,), jnp.int32),
          pltpu.VMEM((CH + BATCH + L,), jnp.int32),
          pltpu.VMEM((CH + BATCH + L,), jnp.float32),
          pltpu.VMEM((BATCH, F), jnp.float32),
          pltpu.VMEM((RPT * F,), jnp.float32),
          pltpu.VMEM((NPAD,), jnp.float32),
          pltpu.VMEM((RPT + L,), jnp.float32),
          pltpu.VMEM((F,), jnp.float32),
          pltpu.SemaphoreType.DMA,
      ],
  )(table, src, dst, w, ns, nd, bias)


def _mlp_body(x_ref, w1_ref, b1_ref, w2_ref, o_ref):
  x = x_ref[...]
  a = jnp.dot(x, w1_ref[...], preferred_element_type=jnp.float32,
              precision=lax.Precision.HIGHEST) + b1_ref[...]
  h = jnp.maximum(a, 0.0)
  o_ref[...] = jnp.dot(h, w2_ref[...], preferred_element_type=jnp.float32,
                       precision=lax.Precision.HIGHEST)


def _mlp_call(x, w1, b1, w2):
  blk = 512
  return pl.pallas_call(
      _mlp_body,
      grid=(NPAD // blk,),
      in_specs=[
          pl.BlockSpec((blk, IN_F), lambda i: (i, 0)),
          pl.BlockSpec((IN_F, HID_F), lambda i: (0, 0)),
          pl.BlockSpec((1, HID_F), lambda i: (0, 0)),
          pl.BlockSpec((HID_F, OUT_F), lambda i: (0, 0)),
      ],
      out_specs=pl.BlockSpec((blk, OUT_F), lambda i: (i, 0)),
      out_shape=jax.ShapeDtypeStruct((NPAD, OUT_F), jnp.float32),
  )(x, w1, b1.reshape(1, HID_F), w2)


@jax.jit
def kernel(node_feats, edge_index, edge_weight, W1, b1, W2, b2):
  src = edge_index[0].astype(jnp.int32)
  dst = edge_index[1].astype(jnp.int32)
  w = edge_weight.astype(jnp.float32)

  od, idg = _deg_call(src, dst)
  ns, nd = _norm_call(od, idg)

  nf_pad = jnp.pad(node_feats, ((0, NPAD - N_NODES), (0, 0)))
  zeros_f = jnp.zeros((F,), jnp.float32)
  agg1 = _scatter_call(nf_pad, src, dst, w, ns, nd, zeros_f)
  t = _mlp_call(agg1, W1, b1, W2)
  out = _scatter_call(t, src, dst, w, ns, nd, b2)
  return out[:N_NODES]


# speedup vs baseline: 2.3245x; 1.9599x over previous
import functools

import jax
import jax.numpy as jnp
from jax import lax
from jax.experimental import pallas as pl
from jax.experimental.pallas import tpu as pltpu
from jax.experimental.pallas import tpu_sc as plsc

N_NODES = 10000
N_EDGES = 160000
F = 256
IN_F = 256
HID_F = 512
OUT_F = 256

NC = 2
NS = 16
NW = NC * NS
L = 16

NPAD = 10240
RPT = NPAD // NW
BATCH = 64
CH = 1600
NCHUNK = N_EDGES // CH
assert CH % L == 0 and N_EDGES % CH == 0 and NCHUNK % 2 == 0
CBUF = CH + BATCH + L
EPW = N_EDGES // NW

@functools.cache
def _mesh():
  return plsc.VectorSubcoreMesh(
      core_axis_name="c", subcore_axis_name="s", num_cores=NC, num_subcores=NS
  )


_SC_PARAMS = pltpu.CompilerParams(needs_layout_passes=False)


def _zero_vecs(ref, n16, dtype):
  z = jnp.zeros((L,), dtype)

  def body(i, _):
    ref[pl.ds(i * L, L)] = z
    return 0

  lax.fori_loop(0, n16, body, 0)


def _deg_body(src, dst, od, idg, sbuf, dbuf, hs, hd, tmp, acc_s, acc_d,
              sh_s, sh_d):
  c = lax.axis_index("c")
  s = lax.axis_index("s")
  wid = c * NS + s
  e0 = wid * EPW

  pltpu.sync_copy(src.at[pl.ds(e0, EPW)], sbuf.at[pl.ds(0, EPW)])
  pltpu.sync_copy(dst.at[pl.ds(e0, EPW)], dbuf.at[pl.ds(0, EPW)])

  _zero_vecs(hs, NPAD // L, jnp.int32)
  _zero_vecs(hd, NPAD // L, jnp.int32)

  ones_i = jnp.ones((L,), jnp.int32)
  nfull = EPW // L
  rem = EPW - nfull * L

  def inc(v, _):
    ks = sbuf[pl.ds(v * L, L)]
    kd = dbuf[pl.ds(v * L, L)]
    plsc.addupdate_scatter(hs, [ks], ones_i)
    plsc.addupdate_scatter(hd, [kd], ones_i)
    return 0

  lax.fori_loop(0, nfull, inc, 0)
  if rem:
    mtail = jnp.arange(L, dtype=jnp.int32) < rem
    sbuf[pl.ds(EPW, L)] = jnp.zeros((L,), jnp.int32)
    dbuf[pl.ds(EPW, L)] = jnp.zeros((L,), jnp.int32)
    ks = sbuf[pl.ds(nfull * L, L)]
    kd = dbuf[pl.ds(nfull * L, L)]
    plsc.addupdate_scatter(hs, [ks], ones_i, mask=mtail)
    plsc.addupdate_scatter(hd, [kd], ones_i, mask=mtail)

  pltpu.sync_copy(hs, sh_s.at[s])
  pltpu.sync_copy(hd, sh_d.at[s])
  plsc.subcore_barrier()

  col0 = s * (NPAD // NS)
  nv = (NPAD // NS) // L
  _zero_vecs(acc_s, nv, jnp.int32)
  _zero_vecs(acc_d, nv, jnp.int32)

  def addv(accref):
    def body(i, _):
      accref[pl.ds(i * L, L)] = accref[pl.ds(i * L, L)] + tmp[pl.ds(i * L, L)]
      return 0
    lax.fori_loop(0, nv, body, 0)

  for k in range(NS):
    pltpu.sync_copy(sh_s.at[k, pl.ds(col0, NPAD // NS)], tmp)
    addv(acc_s)
    pltpu.sync_copy(sh_d.at[k, pl.ds(col0, NPAD // NS)], tmp)
    addv(acc_d)

  pltpu.sync_copy(acc_s, od.at[c, pl.ds(col0, NPAD // NS)])
  pltpu.sync_copy(acc_d, idg.at[c, pl.ds(col0, NPAD // NS)])


@functools.partial(jax.jit, static_argnums=())
def _deg_call(src, dst):
  return pl.kernel(
      _deg_body,
      out_type=(
          jax.ShapeDtypeStruct((NC, NPAD), jnp.int32),
          jax.ShapeDtypeStruct((NC, NPAD), jnp.int32),
      ),
      mesh=_mesh(),
      compiler_params=_SC_PARAMS,
      scratch_types=[
          pltpu.VMEM((EPW + L,), jnp.int32),
          pltpu.VMEM((EPW + L,), jnp.int32),
          pltpu.VMEM((NPAD,), jnp.int32),
          pltpu.VMEM((NPAD,), jnp.int32),
          pltpu.VMEM((NPAD // NS,), jnp.int32),
          pltpu.VMEM((NPAD // NS,), jnp.int32),
          pltpu.VMEM((NPAD // NS,), jnp.int32),
          pltpu.VMEM_SHARED((NS, NPAD), jnp.int32),
          pltpu.VMEM_SHARED((NS, NPAD), jnp.int32),
      ],
  )(src, dst)


def _norm_body(od_ref, id_ref, ns_ref, nd_ref):
  do = (od_ref[0] + od_ref[1]).astype(jnp.float32)
  di = (id_ref[0] + id_ref[1]).astype(jnp.float32)
  ns_ref[...] = lax.rsqrt(jnp.maximum(do, 1.0))
  nd_ref[...] = lax.rsqrt(jnp.maximum(di, 1.0))


def _norm_call(od, idg):
  out = pl.pallas_call(
      _norm_body,
      out_shape=(
          jax.ShapeDtypeStruct((NPAD // 128, 128), jnp.float32),
          jax.ShapeDtypeStruct((NPAD // 128, 128), jnp.float32),
      ),
  )(od.reshape(NC, NPAD // 128, 128), idg.reshape(NC, NPAD // 128, 128))
  return out[0].reshape(NPAD), out[1].reshape(NPAD)


def _scatter_body(table, edata, ns, nd, bias, out,
                  ebuf0, ebuf1, csrc, cdst, cw, rows, acc,
                  nsv, ndv, biasv, sem0, sem1, semg):
  c = lax.axis_index("c")
  s = lax.axis_index("s")
  wid = c * NS + s
  lo = wid * RPT

  pltpu.sync_copy(ns, nsv)
  pltpu.sync_copy(nd.at[pl.ds(wid * RPT, RPT)], ndv.at[pl.ds(0, RPT)])
  pltpu.sync_copy(bias, biasv)

  _zero_vecs(acc, RPT * F // L, jnp.float32)

  padg = lax.rem(wid * 313 + 17, N_NODES)
  pg = jnp.full((L,), 0, jnp.int32) + padg
  zi = jnp.zeros((L,), jnp.int32)
  zf = jnp.zeros((L,), jnp.float32)

  def pre(i, _):
    csrc[pl.ds(i * L, L)] = pg
    cdst[pl.ds(i * L, L)] = zi
    cw[pl.ds(i * L, L)] = zf
    return 0

  lax.fori_loop(0, CBUF // L, pre, 0)

  def chunk_copies(k, ebuf, sem, make):
    e0 = lax.rem(k, NCHUNK) * CH
    mk = pltpu.make_async_copy if make else pltpu.async_copy
    return [
        mk(edata.at[pl.ds(a * N_EDGES + e0, CH)],
           ebuf.at[pl.ds(a * CH, CH)], sem)
        for a in range(3)
    ]

  def issue_chunk(k, ebuf, sem):
    chunk_copies(k, ebuf, sem, make=False)

  def wait_chunk(k, ebuf, sem):
    for cp in chunk_copies(k, ebuf, sem, make=True):
      cp.wait()

  def compress(ebuf, wpos):
    def comp(v, cnt):
      ks = ebuf[pl.ds(v * L, L)]
      kd = ebuf[pl.ds(CH + v * L, L)]
      kw = plsc.bitcast(ebuf[pl.ds(2 * CH + v * L, L)], jnp.float32)
      m = (kd >= lo) & (kd < lo + RPT)
      ld = kd - lo
      plsc.store_compressed(csrc.at[pl.ds(cnt, L)], ks, mask=m)
      plsc.store_compressed(cdst.at[pl.ds(cnt, L)], ld, mask=m)
      plsc.store_compressed(cw.at[pl.ds(cnt, L)], kw, mask=m)
      pc = plsc.all_reduce_population_count(m)
      return cnt + pc[0]

    return lax.fori_loop(0, CH // L, comp, wpos)

  def run_batch(b):
    copies = []
    for j in range(BATCH // L):
      cs = csrc[pl.ds(b * BATCH + j * L, L)]
      nsg = plsc.load_gather(nsv, [cs])
      cw[pl.ds(b * BATCH + j * L, L)] = cw[pl.ds(b * BATCH + j * L, L)] * nsg
      copies.append(
          pltpu.async_copy(table.at[cs], rows.at[pl.ds(j * L, L)], semg))
    for j in range(BATCH // L):
      copies[j].wait()

      def srow(i, _):
        ws = cw[pl.ds(b * BATCH + j * L + i, L)][0]
        ld = cdst[pl.ds(b * BATCH + j * L + i, L)][0]
        base = ld * F
        for jf in range(F // L):
          v = rows[j * L + i, pl.ds(jf * L, L)] * ws
          plsc.addupdate(acc.at[pl.ds(base + jf * L, L)], v)
        return 0

      lax.fori_loop(0, L, srow, 0)

  def process(ebuf, wpos):
    wpos = compress(ebuf, wpos)
    nbf = wpos // BATCH

    def batch_loop(b, _):
      run_batch(b)
      return 0

    lax.fori_loop(0, nbf, batch_loop, 0)
    base = nbf * BATCH
    for j in range(BATCH // L):
      csrc[pl.ds(j * L, L)] = csrc[pl.ds(base + j * L, L)]
      cdst[pl.ds(j * L, L)] = cdst[pl.ds(base + j * L, L)]
      cw[pl.ds(j * L, L)] = cw[pl.ds(base + j * L, L)]
    return wpos - base

  issue_chunk(0, ebuf0, sem0)
  wait_chunk(0, ebuf0, sem0)
  issue_chunk(1, ebuf1, sem1)

  def gbody(g, wpos):
    k0 = 2 * g
    wpos = process(ebuf0, wpos)
    wait_chunk(k0 + 1, ebuf1, sem1)
    issue_chunk(k0 + 2, ebuf0, sem0)
    wpos = process(ebuf1, wpos)
    wait_chunk(k0 + 2, ebuf0, sem0)
    issue_chunk(k0 + 3, ebuf1, sem1)
    return wpos

  wpos = lax.fori_loop(0, NCHUNK // 2, gbody, 0)
  wait_chunk(1, ebuf1, sem1)

  for j in range(BATCH // L):
    cw[pl.ds(wpos + j * L, L)] = zf
  run_batch(0)

  for blk in range(RPT // BATCH):
    def drow(r, _):
      nds = ndv[pl.ds(blk * BATCH + r, L)][0]
      base = (blk * BATCH + r) * F
      for j in range(F // L):
        v = acc[pl.ds(base + j * L, L)]
        rows[r, pl.ds(j * L, L)] = v * nds + biasv[pl.ds(j * L, L)]
      return 0

    lax.fori_loop(0, BATCH, drow, 0)
    pltpu.sync_copy(rows, out.at[pl.ds(lo + blk * BATCH, BATCH)])


def _scatter_call(table, edata, ns, nd, bias):
  return pl.kernel(
      _scatter_body,
      out_type=jax.ShapeDtypeStruct((NPAD, F), jnp.float32),
      mesh=_mesh(),
      compiler_params=_SC_PARAMS,
      scratch_types=[
          pltpu.VMEM((3 * CH,), jnp.int32),
          pltpu.VMEM((3 * CH,), jnp.int32),
          pltpu.VMEM((CBUF,), jnp.int32),
          pltpu.VMEM((CBUF,), jnp.int32),
          pltpu.VMEM((CBUF,), jnp.float32),
          pltpu.VMEM((BATCH, F), jnp.float32),
          pltpu.VMEM((RPT * F,), jnp.float32),
          pltpu.VMEM((NPAD,), jnp.float32),
          pltpu.VMEM((RPT + L,), jnp.float32),
          pltpu.VMEM((F,), jnp.float32),
          pltpu.SemaphoreType.DMA,
          pltpu.SemaphoreType.DMA,
          pltpu.SemaphoreType.DMA,
      ],
  )(table, edata, ns, nd, bias)


def _mlp_body(x_ref, w1_ref, b1_ref, w2_ref, o_ref):
  x = x_ref[...]
  a = jnp.dot(x, w1_ref[...], preferred_element_type=jnp.float32,
              precision=lax.Precision.HIGHEST) + b1_ref[...]
  h = jnp.maximum(a, 0.0)
  o_ref[...] = jnp.dot(h, w2_ref[...], preferred_element_type=jnp.float32,
                       precision=lax.Precision.HIGHEST)


def _mlp_call(x, w1, b1, w2):
  blk = 512
  return pl.pallas_call(
      _mlp_body,
      grid=(NPAD // blk,),
      in_specs=[
          pl.BlockSpec((blk, IN_F), lambda i: (i, 0)),
          pl.BlockSpec((IN_F, HID_F), lambda i: (0, 0)),
          pl.BlockSpec((1, HID_F), lambda i: (0, 0)),
          pl.BlockSpec((HID_F, OUT_F), lambda i: (0, 0)),
      ],
      out_specs=pl.BlockSpec((blk, OUT_F), lambda i: (i, 0)),
      out_shape=jax.ShapeDtypeStruct((NPAD, OUT_F), jnp.float32),
  )(x, w1, b1.reshape(1, HID_F), w2)


@jax.jit
def kernel(node_feats, edge_index, edge_weight, W1, b1, W2, b2):
  src = edge_index[0].astype(jnp.int32)
  dst = edge_index[1].astype(jnp.int32)
  w = edge_weight.astype(jnp.float32)

  od, idg = _deg_call(src, dst)
  ns, nd = _norm_call(od, idg)

  edata = jnp.concatenate(
      [src, dst, jax.lax.bitcast_convert_type(w, jnp.int32)], axis=0)
  nf_pad = jnp.pad(node_feats, ((0, NPAD - N_NODES), (0, 0)))
  zeros_f = jnp.zeros((F,), jnp.float32)
  agg1 = _scatter_call(nf_pad, edata, ns, nd, zeros_f)
  t = _mlp_call(agg1, W1, b1, W2)
  out = _scatter_call(t, edata, ns, nd, b2)
  return out[:N_NODES]


# speedup vs baseline: 2.4400x; 1.0497x over previous
import functools

import jax
import jax.numpy as jnp
from jax import lax
from jax.experimental import pallas as pl
from jax.experimental.pallas import tpu as pltpu
from jax.experimental.pallas import tpu_sc as plsc

N_NODES = 10000
N_EDGES = 160000
F = 256
IN_F = 256
HID_F = 512
OUT_F = 256

NC = 2
NS = 16
NW = NC * NS
L = 16

NPAD = 10240
RPT = NPAD // NW
BATCH = 64
CH = 1600
NCHUNK = N_EDGES // CH
assert CH % L == 0 and N_EDGES % CH == 0 and NCHUNK % 2 == 0
NH = CH // L // 2
CBUF2 = CH // 2 + BATCH + L
CBUF = 2 * CBUF2 + L
EPW = N_EDGES // NW

@functools.cache
def _mesh():
  return plsc.VectorSubcoreMesh(
      core_axis_name="c", subcore_axis_name="s", num_cores=NC, num_subcores=NS
  )


_SC_PARAMS = pltpu.CompilerParams(needs_layout_passes=False)


def _zero_vecs(ref, n16, dtype):
  z = jnp.zeros((L,), dtype)

  def body(i, _):
    ref[pl.ds(i * L, L)] = z
    return 0

  lax.fori_loop(0, n16, body, 0)


def _deg_body(src, dst, od, idg, sbuf, dbuf, hs, hd, tmp, acc_s, acc_d,
              sh_s, sh_d):
  c = lax.axis_index("c")
  s = lax.axis_index("s")
  wid = c * NS + s
  e0 = wid * EPW

  pltpu.sync_copy(src.at[pl.ds(e0, EPW)], sbuf.at[pl.ds(0, EPW)])
  pltpu.sync_copy(dst.at[pl.ds(e0, EPW)], dbuf.at[pl.ds(0, EPW)])

  _zero_vecs(hs, NPAD // L, jnp.int32)
  _zero_vecs(hd, NPAD // L, jnp.int32)

  ones_i = jnp.ones((L,), jnp.int32)
  nfull = EPW // L
  rem = EPW - nfull * L

  def inc(v, _):
    ks = sbuf[pl.ds(v * L, L)]
    kd = dbuf[pl.ds(v * L, L)]
    plsc.addupdate_scatter(hs, [ks], ones_i)
    plsc.addupdate_scatter(hd, [kd], ones_i)
    return 0

  lax.fori_loop(0, nfull, inc, 0)
  if rem:
    mtail = jnp.arange(L, dtype=jnp.int32) < rem
    sbuf[pl.ds(EPW, L)] = jnp.zeros((L,), jnp.int32)
    dbuf[pl.ds(EPW, L)] = jnp.zeros((L,), jnp.int32)
    ks = sbuf[pl.ds(nfull * L, L)]
    kd = dbuf[pl.ds(nfull * L, L)]
    plsc.addupdate_scatter(hs, [ks], ones_i, mask=mtail)
    plsc.addupdate_scatter(hd, [kd], ones_i, mask=mtail)

  pltpu.sync_copy(hs, sh_s.at[s])
  pltpu.sync_copy(hd, sh_d.at[s])
  plsc.subcore_barrier()

  col0 = s * (NPAD // NS)
  nv = (NPAD // NS) // L
  _zero_vecs(acc_s, nv, jnp.int32)
  _zero_vecs(acc_d, nv, jnp.int32)

  def addv(accref):
    def body(i, _):
      accref[pl.ds(i * L, L)] = accref[pl.ds(i * L, L)] + tmp[pl.ds(i * L, L)]
      return 0
    lax.fori_loop(0, nv, body, 0)

  for k in range(NS):
    pltpu.sync_copy(sh_s.at[k, pl.ds(col0, NPAD // NS)], tmp)
    addv(acc_s)
    pltpu.sync_copy(sh_d.at[k, pl.ds(col0, NPAD // NS)], tmp)
    addv(acc_d)

  pltpu.sync_copy(acc_s, od.at[c, pl.ds(col0, NPAD // NS)])
  pltpu.sync_copy(acc_d, idg.at[c, pl.ds(col0, NPAD // NS)])


@functools.partial(jax.jit, static_argnums=())
def _deg_call(src, dst):
  return pl.kernel(
      _deg_body,
      out_type=(
          jax.ShapeDtypeStruct((NC, NPAD), jnp.int32),
          jax.ShapeDtypeStruct((NC, NPAD), jnp.int32),
      ),
      mesh=_mesh(),
      compiler_params=_SC_PARAMS,
      scratch_types=[
          pltpu.VMEM((EPW + L,), jnp.int32),
          pltpu.VMEM((EPW + L,), jnp.int32),
          pltpu.VMEM((NPAD,), jnp.int32),
          pltpu.VMEM((NPAD,), jnp.int32),
          pltpu.VMEM((NPAD // NS,), jnp.int32),
          pltpu.VMEM((NPAD // NS,), jnp.int32),
          pltpu.VMEM((NPAD // NS,), jnp.int32),
          pltpu.VMEM_SHARED((NS, NPAD), jnp.int32),
          pltpu.VMEM_SHARED((NS, NPAD), jnp.int32),
      ],
  )(src, dst)


def _norm_body(od_ref, id_ref, ns_ref, nd_ref):
  do = (od_ref[0] + od_ref[1]).astype(jnp.float32)
  di = (id_ref[0] + id_ref[1]).astype(jnp.float32)
  ns_ref[...] = lax.rsqrt(jnp.maximum(do, 1.0))
  nd_ref[...] = lax.rsqrt(jnp.maximum(di, 1.0))


def _norm_call(od, idg):
  out = pl.pallas_call(
      _norm_body,
      out_shape=(
          jax.ShapeDtypeStruct((NPAD // 128, 128), jnp.float32),
          jax.ShapeDtypeStruct((NPAD // 128, 128), jnp.float32),
      ),
  )(od.reshape(NC, NPAD // 128, 128), idg.reshape(NC, NPAD // 128, 128))
  return out[0].reshape(NPAD), out[1].reshape(NPAD)


def _scatter_body(table, edata, ns, nd, bias, out,
                  ebuf0, ebuf1, csrc, cdst, cw, rows, acc,
                  nsv, ndv, biasv, sem0, sem1, semg):
  c = lax.axis_index("c")
  s = lax.axis_index("s")
  wid = c * NS + s
  lo = wid * RPT

  pltpu.sync_copy(ns, nsv)
  pltpu.sync_copy(nd.at[pl.ds(wid * RPT, RPT)], ndv.at[pl.ds(0, RPT)])
  pltpu.sync_copy(bias, biasv)

  _zero_vecs(acc, RPT * F // L, jnp.float32)

  padg = lax.rem(wid * 313 + 17, N_NODES)
  pg = jnp.full((L,), 0, jnp.int32) + padg
  zi = jnp.zeros((L,), jnp.int32)
  zf = jnp.zeros((L,), jnp.float32)

  def pre(i, _):
    csrc[pl.ds(i * L, L)] = pg
    cdst[pl.ds(i * L, L)] = zi
    cw[pl.ds(i * L, L)] = zf
    return 0

  lax.fori_loop(0, CBUF // L, pre, 0)

  def chunk_copies(k, ebuf, sem, make):
    e0 = lax.rem(k, NCHUNK) * CH
    mk = pltpu.make_async_copy if make else pltpu.async_copy
    return [
        mk(edata.at[pl.ds(a * N_EDGES + e0, CH)],
           ebuf.at[pl.ds(a * CH, CH)], sem)
        for a in range(3)
    ]

  def issue_chunk(k, ebuf, sem):
    chunk_copies(k, ebuf, sem, make=False)

  def wait_chunk(k, ebuf, sem):
    for cp in chunk_copies(k, ebuf, sem, make=True):
      cp.wait()

  def compress(ebuf, carry):
    def comp(v, cc):
      cnts = list(cc)
      for st in range(2):
        vv = st * NH + v
        ks = ebuf[pl.ds(vv * L, L)]
        kd = ebuf[pl.ds(CH + vv * L, L)]
        kw = plsc.bitcast(ebuf[pl.ds(2 * CH + vv * L, L)], jnp.float32)
        m = (kd >= lo) & (kd < lo + RPT)
        ld = kd - lo
        off = st * CBUF2 + cnts[st]
        plsc.store_compressed(csrc.at[pl.ds(off, L)], ks, mask=m)
        plsc.store_compressed(cdst.at[pl.ds(off, L)], ld, mask=m)
        plsc.store_compressed(cw.at[pl.ds(off, L)], kw, mask=m)
        pc = plsc.all_reduce_population_count(m)
        cnts[st] = cnts[st] + pc[0]
      return tuple(cnts)

    return lax.fori_loop(0, NH, comp, carry)

  def run_batch(start):
    copies = []
    for j in range(BATCH // L):
      cs = csrc[pl.ds(start + j * L, L)]
      nsg = plsc.load_gather(nsv, [cs])
      cw[pl.ds(start + j * L, L)] = cw[pl.ds(start + j * L, L)] * nsg
      copies.append(
          pltpu.async_copy(table.at[cs], rows.at[pl.ds(j * L, L)], semg))
    for j in range(BATCH // L):
      copies[j].wait()

      def srow4(q, _):
        cwv = cw[pl.ds(start + j * L + q * 4, L)]
        cdv = cdst[pl.ds(start + j * L + q * 4, L)]
        basev = cdv * F
        for u in range(4):
          ws = cwv[u]
          bb = basev[u]
          r = j * L + q * 4 + u
          for jf in range(F // L):
            v = rows[r, pl.ds(jf * L, L)] * ws
            plsc.addupdate(acc.at[pl.ds(bb + jf * L, L)], v)
        return 0

      lax.fori_loop(0, L // 4, srow4, 0)

  def stream_batches(st, wpos):
    off = st * CBUF2
    nbf = wpos // BATCH

    def batch_loop(b, _):
      run_batch(off + b * BATCH)
      return 0

    lax.fori_loop(0, nbf, batch_loop, 0)
    base = off + nbf * BATCH
    for j in range(BATCH // L):
      csrc[pl.ds(off + j * L, L)] = csrc[pl.ds(base + j * L, L)]
      cdst[pl.ds(off + j * L, L)] = cdst[pl.ds(base + j * L, L)]
      cw[pl.ds(off + j * L, L)] = cw[pl.ds(base + j * L, L)]
    return wpos - nbf * BATCH

  def process(ebuf, carry):
    wa, wb = compress(ebuf, carry)
    wa = stream_batches(0, wa)
    wb = stream_batches(1, wb)
    return wa, wb

  issue_chunk(0, ebuf0, sem0)
  wait_chunk(0, ebuf0, sem0)
  issue_chunk(1, ebuf1, sem1)

  def gbody(g, carry):
    k0 = 2 * g
    carry = process(ebuf0, carry)
    wait_chunk(k0 + 1, ebuf1, sem1)
    issue_chunk(k0 + 2, ebuf0, sem0)
    carry = process(ebuf1, carry)
    wait_chunk(k0 + 2, ebuf0, sem0)
    issue_chunk(k0 + 3, ebuf1, sem1)
    return carry

  wa, wb = lax.fori_loop(0, NCHUNK // 2, gbody, (0, 0))
  wait_chunk(1, ebuf1, sem1)

  for j in range(BATCH // L):
    cw[pl.ds(wa + j * L, L)] = zf
    cw[pl.ds(CBUF2 + wb + j * L, L)] = zf
  run_batch(0)
  run_batch(CBUF2)

  for blk in range(RPT // BATCH):
    def drow(r, _):
      nds = ndv[pl.ds(blk * BATCH + r, L)][0]
      base = (blk * BATCH + r) * F
      for j in range(F // L):
        v = acc[pl.ds(base + j * L, L)]
        rows[r, pl.ds(j * L, L)] = v * nds + biasv[pl.ds(j * L, L)]
      return 0

    lax.fori_loop(0, BATCH, drow, 0)
    pltpu.sync_copy(rows, out.at[pl.ds(lo + blk * BATCH, BATCH)])


def _scatter_call(table, edata, ns, nd, bias):
  return pl.kernel(
      _scatter_body,
      out_type=jax.ShapeDtypeStruct((NPAD, F), jnp.float32),
      mesh=_mesh(),
      compiler_params=_SC_PARAMS,
      scratch_types=[
          pltpu.VMEM((3 * CH,), jnp.int32),
          pltpu.VMEM((3 * CH,), jnp.int32),
          pltpu.VMEM((CBUF,), jnp.int32),
          pltpu.VMEM((CBUF,), jnp.int32),
          pltpu.VMEM((CBUF,), jnp.float32),
          pltpu.VMEM((BATCH, F), jnp.float32),
          pltpu.VMEM((RPT * F,), jnp.float32),
          pltpu.VMEM((NPAD,), jnp.float32),
          pltpu.VMEM((RPT + L,), jnp.float32),
          pltpu.VMEM((F,), jnp.float32),
          pltpu.SemaphoreType.DMA,
          pltpu.SemaphoreType.DMA,
          pltpu.SemaphoreType.DMA,
      ],
  )(table, edata, ns, nd, bias)


def _mlp_body(x_ref, w1_ref, b1_ref, w2_ref, o_ref):
  x = x_ref[...]
  a = jnp.dot(x, w1_ref[...], preferred_element_type=jnp.float32,
              precision=lax.Precision.HIGHEST) + b1_ref[...]
  h = jnp.maximum(a, 0.0)
  o_ref[...] = jnp.dot(h, w2_ref[...], preferred_element_type=jnp.float32,
                       precision=lax.Precision.HIGHEST)


def _mlp_call(x, w1, b1, w2):
  blk = 512
  return pl.pallas_call(
      _mlp_body,
      grid=(NPAD // blk,),
      in_specs=[
          pl.BlockSpec((blk, IN_F), lambda i: (i, 0)),
          pl.BlockSpec((IN_F, HID_F), lambda i: (0, 0)),
          pl.BlockSpec((1, HID_F), lambda i: (0, 0)),
          pl.BlockSpec((HID_F, OUT_F), lambda i: (0, 0)),
      ],
      out_specs=pl.BlockSpec((blk, OUT_F), lambda i: (i, 0)),
      out_shape=jax.ShapeDtypeStruct((NPAD, OUT_F), jnp.float32),
  )(x, w1, b1.reshape(1, HID_F), w2)


@jax.jit
def kernel(node_feats, edge_index, edge_weight, W1, b1, W2, b2):
  src = edge_index[0].astype(jnp.int32)
  dst = edge_index[1].astype(jnp.int32)
  w = edge_weight.astype(jnp.float32)

  od, idg = _deg_call(src, dst)
  ns, nd = _norm_call(od, idg)

  edata = jnp.concatenate(
      [src, dst, jax.lax.bitcast_convert_type(w, jnp.int32)], axis=0)
  nf_pad = jnp.pad(node_feats, ((0, NPAD - N_NODES), (0, 0)))
  zeros_f = jnp.zeros((F,), jnp.float32)
  agg1 = _scatter_call(nf_pad, edata, ns, nd, zeros_f)
  t = _mlp_call(agg1, W1, b1, W2)
  out = _scatter_call(t, edata, ns, nd, b2)
  return out[:N_NODES]


# speedup vs baseline: 2.4468x; 1.0028x over previous
import functools

import jax
import jax.numpy as jnp
from jax import lax
from jax.experimental import pallas as pl
from jax.experimental.pallas import tpu as pltpu
from jax.experimental.pallas import tpu_sc as plsc

N_NODES = 10000
N_EDGES = 160000
F = 256
IN_F = 256
HID_F = 512
OUT_F = 256

NC = 2
NS = 16
NW = NC * NS
L = 16

NPAD = 10240
RPT = NPAD // NW
BATCH = 64
CH = 1600
NCHUNK = N_EDGES // CH
assert CH % L == 0 and N_EDGES % CH == 0 and NCHUNK % 2 == 0
NH = CH // L // 2
CBUF2 = CH // 2 + BATCH + L
CBUF = 2 * CBUF2 + L
EPW = N_EDGES // NW

@functools.cache
def _mesh():
  return plsc.VectorSubcoreMesh(
      core_axis_name="c", subcore_axis_name="s", num_cores=NC, num_subcores=NS
  )


_SC_PARAMS = pltpu.CompilerParams(needs_layout_passes=False)


def _zero_vecs(ref, n16, dtype):
  z = jnp.zeros((L,), dtype)

  def body(i, _):
    ref[pl.ds(i * L, L)] = z
    return 0

  lax.fori_loop(0, n16, body, 0)


def _deg_body(src, dst, od, idg, sbuf, dbuf, hs, hd, tmp, acc_s, acc_d,
              sh_s, sh_d):
  c = lax.axis_index("c")
  s = lax.axis_index("s")
  wid = c * NS + s
  e0 = wid * EPW

  pltpu.sync_copy(src.at[pl.ds(e0, EPW)], sbuf.at[pl.ds(0, EPW)])
  pltpu.sync_copy(dst.at[pl.ds(e0, EPW)], dbuf.at[pl.ds(0, EPW)])

  _zero_vecs(hs, NPAD // L, jnp.int32)
  _zero_vecs(hd, NPAD // L, jnp.int32)

  ones_i = jnp.ones((L,), jnp.int32)
  nfull = EPW // L
  rem = EPW - nfull * L

  def inc(v, _):
    ks = sbuf[pl.ds(v * L, L)]
    kd = dbuf[pl.ds(v * L, L)]
    plsc.addupdate_scatter(hs, [ks], ones_i)
    plsc.addupdate_scatter(hd, [kd], ones_i)
    return 0

  lax.fori_loop(0, nfull, inc, 0)
  if rem:
    mtail = jnp.arange(L, dtype=jnp.int32) < rem
    sbuf[pl.ds(EPW, L)] = jnp.zeros((L,), jnp.int32)
    dbuf[pl.ds(EPW, L)] = jnp.zeros((L,), jnp.int32)
    ks = sbuf[pl.ds(nfull * L, L)]
    kd = dbuf[pl.ds(nfull * L, L)]
    plsc.addupdate_scatter(hs, [ks], ones_i, mask=mtail)
    plsc.addupdate_scatter(hd, [kd], ones_i, mask=mtail)

  pltpu.sync_copy(hs, sh_s.at[s])
  pltpu.sync_copy(hd, sh_d.at[s])
  plsc.subcore_barrier()

  col0 = s * (NPAD // NS)
  nv = (NPAD // NS) // L
  _zero_vecs(acc_s, nv, jnp.int32)
  _zero_vecs(acc_d, nv, jnp.int32)

  def addv(accref):
    def body(i, _):
      accref[pl.ds(i * L, L)] = accref[pl.ds(i * L, L)] + tmp[pl.ds(i * L, L)]
      return 0
    lax.fori_loop(0, nv, body, 0)

  for k in range(NS):
    pltpu.sync_copy(sh_s.at[k, pl.ds(col0, NPAD // NS)], tmp)
    addv(acc_s)
    pltpu.sync_copy(sh_d.at[k, pl.ds(col0, NPAD // NS)], tmp)
    addv(acc_d)

  pltpu.sync_copy(acc_s, od.at[c, pl.ds(col0, NPAD // NS)])
  pltpu.sync_copy(acc_d, idg.at[c, pl.ds(col0, NPAD // NS)])


@functools.partial(jax.jit, static_argnums=())
def _deg_call(src, dst):
  return pl.kernel(
      _deg_body,
      out_type=(
          jax.ShapeDtypeStruct((NC, NPAD), jnp.int32),
          jax.ShapeDtypeStruct((NC, NPAD), jnp.int32),
      ),
      mesh=_mesh(),
      compiler_params=_SC_PARAMS,
      scratch_types=[
          pltpu.VMEM((EPW + L,), jnp.int32),
          pltpu.VMEM((EPW + L,), jnp.int32),
          pltpu.VMEM((NPAD,), jnp.int32),
          pltpu.VMEM((NPAD,), jnp.int32),
          pltpu.VMEM((NPAD // NS,), jnp.int32),
          pltpu.VMEM((NPAD // NS,), jnp.int32),
          pltpu.VMEM((NPAD // NS,), jnp.int32),
          pltpu.VMEM_SHARED((NS, NPAD), jnp.int32),
          pltpu.VMEM_SHARED((NS, NPAD), jnp.int32),
      ],
  )(src, dst)


def _norm_body(od_ref, id_ref, ns_ref, nd_ref):
  do = (od_ref[0] + od_ref[1]).astype(jnp.float32)
  di = (id_ref[0] + id_ref[1]).astype(jnp.float32)
  ns_ref[...] = lax.rsqrt(jnp.maximum(do, 1.0))
  nd_ref[...] = lax.rsqrt(jnp.maximum(di, 1.0))


def _norm_call(od, idg):
  out = pl.pallas_call(
      _norm_body,
      out_shape=(
          jax.ShapeDtypeStruct((NPAD // 128, 128), jnp.float32),
          jax.ShapeDtypeStruct((NPAD // 128, 128), jnp.float32),
      ),
  )(od.reshape(NC, NPAD // 128, 128), idg.reshape(NC, NPAD // 128, 128))
  return out[0].reshape(NPAD), out[1].reshape(NPAD)


def _prep_body(src, dst, w, ns, p, we, sbuf, dbuf, wbuf, pbuf, webuf, nsv):
  c = lax.axis_index("c")
  s = lax.axis_index("s")
  wid = c * NS + s
  e0 = wid * EPW

  pltpu.sync_copy(ns, nsv)
  pltpu.sync_copy(src.at[pl.ds(e0, EPW)], sbuf.at[pl.ds(0, EPW)])
  pltpu.sync_copy(dst.at[pl.ds(e0, EPW)], dbuf.at[pl.ds(0, EPW)])
  pltpu.sync_copy(w.at[pl.ds(e0, EPW)], wbuf.at[pl.ds(0, EPW)])
  sbuf[pl.ds(EPW, L)] = jnp.zeros((L,), jnp.int32)
  dbuf[pl.ds(EPW, L)] = jnp.zeros((L,), jnp.int32)

  def body(v, _):
    ks = sbuf[pl.ds(v * L, L)]
    kd = dbuf[pl.ds(v * L, L)]
    kw = wbuf[pl.ds(v * L, L)]
    nsg = plsc.load_gather(nsv, [ks])
    pbuf[pl.ds(v * L, L)] = kd * 16384 + ks
    webuf[pl.ds(v * L, L)] = kw * nsg
    return 0

  lax.fori_loop(0, (EPW + L - 1) // L, body, 0)
  pltpu.sync_copy(pbuf.at[pl.ds(0, EPW)], p.at[pl.ds(e0, EPW)])
  pltpu.sync_copy(webuf.at[pl.ds(0, EPW)], we.at[pl.ds(e0, EPW)])


def _prep_call(src, dst, w, ns):
  return pl.kernel(
      _prep_body,
      out_type=(
          jax.ShapeDtypeStruct((N_EDGES,), jnp.int32),
          jax.ShapeDtypeStruct((N_EDGES,), jnp.float32),
      ),
      mesh=_mesh(),
      compiler_params=_SC_PARAMS,
      scratch_types=[
          pltpu.VMEM((EPW + L,), jnp.int32),
          pltpu.VMEM((EPW + L,), jnp.int32),
          pltpu.VMEM((EPW + L,), jnp.float32),
          pltpu.VMEM((EPW + L,), jnp.int32),
          pltpu.VMEM((EPW + L,), jnp.float32),
          pltpu.VMEM((NPAD,), jnp.float32),
      ],
  )(src, dst, w, ns)


def _scatter_body(table, p, we, nd, bias, out,
                  ep0, ew0, ep1, ew1, cp, cwe, rows, acc,
                  ndv, biasv, sem0, sem1, semg):
  c = lax.axis_index("c")
  s = lax.axis_index("s")
  wid = c * NS + s
  lo = wid * RPT
  plo = lo * 16384
  phi = (lo + RPT) * 16384

  pltpu.sync_copy(nd.at[pl.ds(wid * RPT, RPT)], ndv.at[pl.ds(0, RPT)])
  pltpu.sync_copy(bias, biasv)

  _zero_vecs(acc, RPT * F // L, jnp.float32)

  padg = lax.rem(wid * 313 + 17, N_NODES)
  pg = jnp.full((L,), 0, jnp.int32) + (plo + padg)
  zf = jnp.zeros((L,), jnp.float32)

  def pre(i, _):
    cp[pl.ds(i * L, L)] = pg
    cwe[pl.ds(i * L, L)] = zf
    return 0

  lax.fori_loop(0, CBUF // L, pre, 0)

  def chunk_copies(k, ebp, ebw, sem, make):
    e0 = lax.rem(k, NCHUNK) * CH
    mk = pltpu.make_async_copy if make else pltpu.async_copy
    return [
        mk(p.at[pl.ds(e0, CH)], ebp, sem),
        mk(we.at[pl.ds(e0, CH)], ebw, sem),
    ]

  def issue_chunk(k, ebp, ebw, sem):
    chunk_copies(k, ebp, ebw, sem, make=False)

  def wait_chunk(k, ebp, ebw, sem):
    for cc in chunk_copies(k, ebp, ebw, sem, make=True):
      cc.wait()

  def compress(ebp, ebw, carry):
    def comp(v, cc):
      cnts = list(cc)
      for st in range(2):
        vv = st * NH + v
        kp = ebp[pl.ds(vv * L, L)]
        kw = ebw[pl.ds(vv * L, L)]
        m = (kp >= plo) & (kp < phi)
        off = st * CBUF2 + cnts[st]
        plsc.store_compressed(cp.at[pl.ds(off, L)], kp, mask=m)
        plsc.store_compressed(cwe.at[pl.ds(off, L)], kw, mask=m)
        pc = plsc.all_reduce_population_count(m)
        cnts[st] = cnts[st] + pc[0]
      return tuple(cnts)

    return lax.fori_loop(0, NH, comp, carry)

  def run_batch(start):
    copies = []
    for j in range(BATCH // L):
      cs = cp[pl.ds(start + j * L, L)] & 16383
      copies.append(
          pltpu.async_copy(table.at[cs], rows.at[pl.ds(j * L, L)], semg))
    for j in range(BATCH // L):
      copies[j].wait()

      def srow4(q, _):
        cwv = cwe[pl.ds(start + j * L + q * 4, L)]
        cpv = cp[pl.ds(start + j * L + q * 4, L)]
        basev = (lax.shift_right_logical(cpv, 14) - lo) * F
        for u in range(4):
          ws = cwv[u]
          bb = basev[u]
          r = j * L + q * 4 + u
          for jf in range(F // L):
            v = rows[r, pl.ds(jf * L, L)] * ws
            plsc.addupdate(acc.at[pl.ds(bb + jf * L, L)], v)
        return 0

      lax.fori_loop(0, L // 4, srow4, 0)

  def stream_batches(st, wpos):
    off = st * CBUF2
    nbf = wpos // BATCH

    def batch_loop(b, _):
      run_batch(off + b * BATCH)
      return 0

    lax.fori_loop(0, nbf, batch_loop, 0)
    base = off + nbf * BATCH
    for j in range(BATCH // L):
      cp[pl.ds(off + j * L, L)] = cp[pl.ds(base + j * L, L)]
      cwe[pl.ds(off + j * L, L)] = cwe[pl.ds(base + j * L, L)]
    return wpos - nbf * BATCH

  def process(ebp, ebw, carry):
    wa, wb = compress(ebp, ebw, carry)
    wa = stream_batches(0, wa)
    wb = stream_batches(1, wb)
    return wa, wb

  issue_chunk(0, ep0, ew0, sem0)
  wait_chunk(0, ep0, ew0, sem0)
  issue_chunk(1, ep1, ew1, sem1)

  def gbody(g, carry):
    k0 = 2 * g
    carry = process(ep0, ew0, carry)
    wait_chunk(k0 + 1, ep1, ew1, sem1)
    issue_chunk(k0 + 2, ep0, ew0, sem0)
    carry = process(ep1, ew1, carry)
    wait_chunk(k0 + 2, ep0, ew0, sem0)
    issue_chunk(k0 + 3, ep1, ew1, sem1)
    return carry

  wa, wb = lax.fori_loop(0, NCHUNK // 2, gbody, (0, 0))
  wait_chunk(1, ep1, ew1, sem1)

  for j in range(BATCH // L):
    cwe[pl.ds(wa + j * L, L)] = zf
    cwe[pl.ds(CBUF2 + wb + j * L, L)] = zf
  run_batch(0)
  run_batch(CBUF2)

  for blk in range(RPT // BATCH):
    def drow(r, _):
      nds = ndv[pl.ds(blk * BATCH + r, L)][0]
      base = (blk * BATCH + r) * F
      for j in range(F // L):
        v = acc[pl.ds(base + j * L, L)]
        rows[r, pl.ds(j * L, L)] = v * nds + biasv[pl.ds(j * L, L)]
      return 0

    lax.fori_loop(0, BATCH, drow, 0)
    pltpu.sync_copy(rows, out.at[pl.ds(lo + blk * BATCH, BATCH)])


def _scatter_call(table, p, we, nd, bias):
  return pl.kernel(
      _scatter_body,
      out_type=jax.ShapeDtypeStruct((NPAD, F), jnp.float32),
      mesh=_mesh(),
      compiler_params=_SC_PARAMS,
      scratch_types=[
          pltpu.VMEM((CH,), jnp.int32),
          pltpu.VMEM((CH,), jnp.float32),
          pltpu.VMEM((CH,), jnp.int32),
          pltpu.VMEM((CH,), jnp.float32),
          pltpu.VMEM((CBUF,), jnp.int32),
          pltpu.VMEM((CBUF,), jnp.float32),
          pltpu.VMEM((BATCH, F), jnp.float32),
          pltpu.VMEM((RPT * F,), jnp.float32),
          pltpu.VMEM((RPT + L,), jnp.float32),
          pltpu.VMEM((F,), jnp.float32),
          pltpu.SemaphoreType.DMA,
          pltpu.SemaphoreType.DMA,
          pltpu.SemaphoreType.DMA,
      ],
  )(table, p, we, nd, bias)


def _mlp_body(x_ref, w1_ref, b1_ref, w2_ref, o_ref):
  x = x_ref[...]
  a = jnp.dot(x, w1_ref[...], preferred_element_type=jnp.float32,
              precision=lax.Precision.HIGHEST) + b1_ref[...]
  h = jnp.maximum(a, 0.0)
  o_ref[...] = jnp.dot(h, w2_ref[...], preferred_element_type=jnp.float32,
                       precision=lax.Precision.HIGHEST)


def _mlp_call(x, w1, b1, w2):
  blk = 512
  return pl.pallas_call(
      _mlp_body,
      grid=(NPAD // blk,),
      in_specs=[
          pl.BlockSpec((blk, IN_F), lambda i: (i, 0)),
          pl.BlockSpec((IN_F, HID_F), lambda i: (0, 0)),
          pl.BlockSpec((1, HID_F), lambda i: (0, 0)),
          pl.BlockSpec((HID_F, OUT_F), lambda i: (0, 0)),
      ],
      out_specs=pl.BlockSpec((blk, OUT_F), lambda i: (i, 0)),
      out_shape=jax.ShapeDtypeStruct((NPAD, OUT_F), jnp.float32),
  )(x, w1, b1.reshape(1, HID_F), w2)


@jax.jit
def kernel(node_feats, edge_index, edge_weight, W1, b1, W2, b2):
  src = edge_index[0].astype(jnp.int32)
  dst = edge_index[1].astype(jnp.int32)
  w = edge_weight.astype(jnp.float32)

  od, idg = _deg_call(src, dst)
  ns, nd = _norm_call(od, idg)
  p, we = _prep_call(src, dst, w, ns)

  nf_pad = jnp.pad(node_feats, ((0, NPAD - N_NODES), (0, 0)))
  zeros_f = jnp.zeros((F,), jnp.float32)
  agg1 = _scatter_call(nf_pad, p, we, nd, zeros_f)
  t = _mlp_call(agg1, W1, b1, W2)
  out = _scatter_call(t, p, we, nd, b2)
  return out[:N_NODES]
